# baseline (speedup 1.0000x reference)
import functools

import jax
import jax.numpy as jnp
from jax import lax
from jax.experimental import pallas as pl
from jax.experimental.pallas import tpu as pltpu
from jax.experimental.pallas import tpu_sc as plsc

NND = 10000
NED = 320000
HD = 128
NCORE, NSUB, LN = 2, 16, 16
NREG = NCORE * NSUB
NPADR = 10240
ACCR = 10016
DUMPROW = NND
CH = 192
SL = NED // NREG
BLKE = 2000
PADN = SL + 4 * CH

_mesh = plsc.VectorSubcoreMesh(
    core_axis_name="c", subcore_axis_name="s", num_cores=NCORE, num_subcores=NSUB)
_sc_params = pltpu.CompilerParams(
    needs_layout_passes=False, use_tc_tiling_on_sc=False)


def _lrelu(v):
    return jnp.where(v >= 0, v, 0.01 * v)


@functools.partial(
    pl.kernel,
    out_type=[
        jax.ShapeDtypeStruct((2 * NREG * PADN,), jnp.int32),
        jax.ShapeDtypeStruct((2 * NREG * PADN,), jnp.int32),
        jax.ShapeDtypeStruct((NREG, LN), jnp.int32),
        jax.ShapeDtypeStruct((NREG, 2 * NPADR), jnp.float32),
    ],
    mesh=_mesh,
    scratch_types=[
        pltpu.VMEM((BLKE,), jnp.int32),
        pltpu.VMEM((BLKE,), jnp.int32),
        pltpu.VMEM((BLKE,), jnp.int32),
        pltpu.VMEM((PADN,), jnp.int32),
        pltpu.VMEM((PADN,), jnp.int32),
        pltpu.VMEM((PADN,), jnp.int32),
        pltpu.VMEM((PADN,), jnp.int32),
        pltpu.VMEM((LN,), jnp.int32),
        pltpu.VMEM((2 * NPADR,), jnp.float32),
    ],
    compiler_params=_sc_params,
)
def _sc_part(src_hbm, dst_hbm, typ_hbm, psrc_hbm, pdst_hbm, narr_hbm, cnts_hbm,
             sb, db, tb, cs0, cd0, cs1, cd1, nb, cnt):
    cid = lax.axis_index("c")
    sid = lax.axis_index("s")
    wid = cid * NSUB + sid
    zeros16 = jnp.zeros((LN,), jnp.float32)
    zeros16i = jnp.zeros((LN,), jnp.int32)
    ones16 = jnp.ones((LN,), jnp.float32)
    dump16 = jnp.full((LN,), DUMPROW, jnp.int32)
    iota16 = lax.broadcasted_iota(jnp.int32, (LN,), 0)

    def zc(i, carry):
        cnt[pl.ds(i * LN, LN)] = zeros16
        return carry
    lax.fori_loop(0, 2 * NPADR // LN, zc, 0)

    def blk(b, fills):
        eoff = wid * SL + b * BLKE
        pltpu.sync_copy(src_hbm.at[pl.ds(eoff, BLKE)], sb)
        pltpu.sync_copy(dst_hbm.at[pl.ds(eoff, BLKE)], db)
        pltpu.sync_copy(typ_hbm.at[pl.ds(eoff, BLKE)], tb)

        def grp(g, fills2):
            f0, f1 = fills2
            s16 = sb[pl.ds(g * LN, LN)]
            d16 = db[pl.ds(g * LN, LN)]
            t16 = tb[pl.ds(g * LN, LN)]
            plsc.addupdate_scatter(cnt, [d16 * 2 + t16], ones16)
            m0 = t16 == 0
            m1 = jnp.logical_not(m0)
            plsc.store_compressed(cs0.at[pl.ds(f0, LN)], s16, mask=m0)
            plsc.store_compressed(cd0.at[pl.ds(f0, LN)], d16, mask=m0)
            plsc.store_compressed(cs1.at[pl.ds(f1, LN)], s16, mask=m1)
            plsc.store_compressed(cd1.at[pl.ds(f1, LN)], d16, mask=m1)
            c0 = jnp.sum(m0.astype(jnp.int32))
            return (f0 + c0, f1 + (LN - c0))
        return lax.fori_loop(0, BLKE // LN, grp, fills)
    f0, f1 = lax.fori_loop(0, SL // BLKE, blk, (0, 0))

    for k in range(4 * CH // LN):
        cs0[pl.ds(f0 + k * LN, LN)] = zeros16i
        cd0[pl.ds(f0 + k * LN, LN)] = dump16
        cs1[pl.ds(f1 + k * LN, LN)] = zeros16i
        cd1[pl.ds(f1 + k * LN, LN)] = dump16
    p0 = lax.div(f0 + 4 * CH - 1, 4 * CH)
    p1 = lax.div(f1 + 4 * CH - 1, 4 * CH)
    nb[pl.ds(0, LN)] = jnp.where(iota16 == 0, p0, jnp.where(iota16 == 1, p1, 0))

    pltpu.sync_copy(cs0, psrc_hbm.at[pl.ds(wid * PADN, PADN)])
    pltpu.sync_copy(cd0, pdst_hbm.at[pl.ds(wid * PADN, PADN)])
    pltpu.sync_copy(cs1, psrc_hbm.at[pl.ds((NREG + wid) * PADN, PADN)])
    pltpu.sync_copy(cd1, pdst_hbm.at[pl.ds((NREG + wid) * PADN, PADN)])
    pltpu.sync_copy(nb, narr_hbm.at[wid])
    pltpu.sync_copy(cnt, cnts_hbm.at[wid])


@functools.partial(
    pl.kernel,
    out_type=jax.ShapeDtypeStruct((2 * NPADR, HD), jnp.float32),
    mesh=_mesh,
    scratch_types=[
        pltpu.VMEM_SHARED((ACCR, HD), jnp.float32),
        pltpu.VMEM((LN,), jnp.int32),
        pltpu.VMEM((LN,), jnp.int32),
        pltpu.VMEM((CH,), jnp.int32),
        pltpu.VMEM((CH,), jnp.int32),
        pltpu.VMEM((CH,), jnp.int32),
        pltpu.VMEM((CH,), jnp.int32),
        pltpu.VMEM((CH,), jnp.int32),
        pltpu.VMEM((CH,), jnp.int32),
        pltpu.VMEM((CH,), jnp.int32),
        pltpu.VMEM((CH,), jnp.int32),
        pltpu.VMEM((CH, HD), jnp.float32),
        pltpu.VMEM((CH, HD), jnp.float32),
        pltpu.SemaphoreType.DMA,
        pltpu.SemaphoreType.DMA,
        pltpu.SemaphoreType.DMA,
        pltpu.SemaphoreType.DMA,
        pltpu.SemaphoreType.DMA,
        pltpu.SemaphoreType.DMA,
        pltpu.SemaphoreType.DMA,
        pltpu.SemaphoreType.DMA,
    ],
    compiler_params=_sc_params,
)
def _sc_agg(x_hbm, psrc_hbm, pdst_hbm, narr_hbm, out_hbm,
            accum, nbA, nbB, s0, d0, s1, d1, s2, d2, s3, d3,
            rowsA, rowsB, ix0, ix1, ix2, ix3, gsA, gsB, ssA, ssB):
    cid = lax.axis_index("c")
    sid = lax.axis_index("s")
    zeros16 = jnp.zeros((LN,), jnp.float32)
    dump16 = jnp.full((LN,), DUMPROW, jnp.int32)
    sets = ((s0, d0, ix0), (s1, d1, ix1), (s2, d2, ix2), (s3, d3, ix3))

    def zrow(r, carry):
        for c in range(8):
            rowsA[r, pl.ds(c * LN, LN)] = zeros16
            rowsB[r, pl.ds(c * LN, LN)] = zeros16
        return carry
    lax.fori_loop(0, CH, zrow, 0)
    for v in range(CH // LN):
        d2[pl.ds(v * LN, LN)] = dump16
        d3[pl.ds(v * LN, LN)] = dump16

    ZR = ACCR // NSUB
    def zacc(i, carry):
        pltpu.sync_copy(rowsA, accum.at[pl.ds(sid * ZR + i * CH, CH)])
        return carry
    lax.fori_loop(0, ZR // CH, zacc, 0)
    pltpu.sync_copy(rowsA.at[pl.ds(0, ZR % CH)],
                    accum.at[pl.ds(sid * ZR + (ZR // CH) * CH, ZR % CH)])
    plsc.subcore_barrier()

    pltpu.sync_copy(narr_hbm.at[2 * sid], nbA)
    pltpu.sync_copy(narr_hbm.at[2 * sid + 1], nbB)
    iota16 = lax.broadcasted_iota(jnp.int32, (LN,), 0)
    qA = jnp.sum(jnp.where(iota16 == cid, nbA[pl.ds(0, LN)], 0))
    qB = jnp.sum(jnp.where(iota16 == cid, nbB[pl.ds(0, LN)], 0))
    baseA = (cid * NREG + 2 * sid) * PADN
    baseB = (cid * NREG + 2 * sid + 1) * PADN
    tot = 4 * (qA + qB)

    def off(c):
        return pl.multiple_of(
            jnp.where(c < 4 * qA, baseA + c * CH, baseB + (c - 4 * qA) * CH), CH)

    def offpf(c):
        return pl.multiple_of(
            jnp.where(c < tot,
                      jnp.where(c < 4 * qA, baseA + c * CH,
                                baseB + (c - 4 * qA) * CH),
                      baseA), CH)

    def idx_pair(k, o):
        sb, db, sem = sets[k]
        return (pltpu.make_async_copy(psrc_hbm.at[pl.ds(o, CH)], sb, sem),
                pltpu.make_async_copy(pdst_hbm.at[pl.ds(o, CH)], db, sem))

    def fire_idx(k, o):
        a, b = idx_pair(k, o)
        a.start(); b.start()

    def wait_idx(k, o):
        a, b = idx_pair(k, o)
        a.wait(); b.wait()

    def scat(rows, idx, sem):
        return pltpu.make_async_copy(rows, accum.at[idx], sem)

    def gath(idx, rows, sem):
        return pltpu.make_async_copy(x_hbm.at[idx], rows, sem)

    fire_idx(0, offpf(0))
    fire_idx(1, offpf(1))
    scat(rowsA, d2, ssA).start(add=True)
    scat(rowsB, d3, ssB).start(add=True)

    def quad(g, carry):
        c0 = 4 * g
        wait_idx(0, off(c0))
        scat(rowsA, d2, ssA).wait()
        fire_idx(2, off(c0 + 2))
        gath(s0, rowsA, gsA).start()
        wait_idx(1, off(c0 + 1))
        scat(rowsB, d3, ssB).wait()
        fire_idx(3, off(c0 + 3))
        gath(s1, rowsB, gsB).start()
        gath(s0, rowsA, gsA).wait()
        scat(rowsA, d0, ssA).start(add=True)
        gath(s1, rowsB, gsB).wait()
        scat(rowsB, d1, ssB).start(add=True)
        scat(rowsA, d0, ssA).wait()
        fire_idx(0, offpf(c0 + 4))
        wait_idx(2, off(c0 + 2))
        gath(s2, rowsA, gsA).start()
        scat(rowsB, d1, ssB).wait()
        fire_idx(1, offpf(c0 + 5))
        wait_idx(3, off(c0 + 3))
        gath(s3, rowsB, gsB).start()
        gath(s2, rowsA, gsA).wait()
        scat(rowsA, d2, ssA).start(add=True)
        gath(s3, rowsB, gsB).wait()
        scat(rowsB, d3, ssB).start(add=True)
        return carry
    lax.fori_loop(0, qA + qB, quad, 0)
    scat(rowsA, d2, ssA).wait()
    scat(rowsB, d3, ssB).wait()
    wait_idx(0, offpf(tot))
    wait_idx(1, offpf(tot + 1))
    plsc.subcore_barrier()

    @pl.when(sid < 10)
    def _():
        pltpu.sync_copy(accum.at[pl.ds(sid * 1000, 1000)],
                        out_hbm.at[pl.ds(cid * NPADR + sid * 1000, 1000)])


_BLK = 1000
_GRID = NND // _BLK


def _tc_pre_body(des_r, tw_r, np_r, cp_r, wd_r, wt_r, wn_r, wc_r,
                 bd_r, bt_r, bn_r, bc_r, win_r, bin_r, out_r):
    d = _lrelu(jnp.dot(des_r[...], wd_r[...], preferred_element_type=jnp.float32) + bd_r[...])
    t = _lrelu(jnp.dot(tw_r[...], wt_r[...], preferred_element_type=jnp.float32) + bt_r[...])
    n = _lrelu(jnp.dot(np_r[...], wn_r[...], preferred_element_type=jnp.float32) + bn_r[...])
    c = _lrelu(jnp.dot(cp_r[...], wc_r[...], preferred_element_type=jnp.float32) + bc_r[...])
    x = jnp.concatenate([d, t, n, c], axis=1)
    out_r[...] = _lrelu(jnp.dot(x, win_r[...], preferred_element_type=jnp.float32) + bin_r[...])


def _tc_pre(des, tw, npad, cpad, wd, wt, wn, wc, bd, bt, bn, bc, win, bin_):
    full = lambda s: pl.BlockSpec(s, lambda i: (0, 0))
    rows = lambda w: pl.BlockSpec((_BLK, w), lambda i: (i, 0))
    return pl.pallas_call(
        _tc_pre_body,
        grid=(_GRID,),
        in_specs=[rows(768), rows(768), rows(8), rows(8),
                  full((768, 32)), full((768, 32)), full((8, 32)), full((8, 32)),
                  full((1, 32)), full((1, 32)), full((1, 32)), full((1, 32)),
                  full((HD, HD)), full((1, HD))],
        out_specs=rows(HD),
        out_shape=jax.ShapeDtypeStruct((NND, HD), jnp.float32),
    )(des, tw, npad, cpad, wd, wt, wn, wc, bd, bt, bn, bc, win, bin_)


def _conv_out(x_r, s0_r, s1_r, cnt_r, root_r, r0_r, r1_r, bias_r):
    cnt = jnp.sum(cnt_r[...], axis=0)
    c0 = jnp.maximum(cnt[:, 0:1], 1.0)
    c1 = jnp.maximum(cnt[:, 1:2], 1.0)
    h0 = s0_r[...] / c0
    h1 = s1_r[...] / c1
    return (jnp.dot(x_r[...], root_r[...], preferred_element_type=jnp.float32)
            + bias_r[...]
            + jnp.dot(h0, r0_r[...], preferred_element_type=jnp.float32)
            + jnp.dot(h1, r1_r[...], preferred_element_type=jnp.float32))


def _tc_conv_body(x_r, s0_r, s1_r, cnt_r, root_r, r0_r, r1_r, bias_r, out_r):
    out_r[...] = _conv_out(x_r, s0_r, s1_r, cnt_r, root_r, r0_r, r1_r, bias_r)


def _tc_conv2_body(x_r, s0_r, s1_r, cnt_r, root_r, r0_r, r1_r, bias_r,
                   wo1_r, bo1_r, wo2_r, bo2_r, out_r):
    o = _conv_out(x_r, s0_r, s1_r, cnt_r, root_r, r0_r, r1_r, bias_r)
    y = _lrelu(jnp.dot(o, wo1_r[...], preferred_element_type=jnp.float32) + bo1_r[...])
    out_r[...] = jnp.dot(y, wo2_r[...], preferred_element_type=jnp.float32) + bo2_r[...]


def _tc_conv(x, s0, s1, cnt, root, r0, r1, bias):
    full = lambda s: pl.BlockSpec(s, lambda i: (0, 0))
    rows = lambda w: pl.BlockSpec((_BLK, w), lambda i: (i, 0))
    return pl.pallas_call(
        _tc_conv_body,
        grid=(_GRID,),
        in_specs=[rows(HD), rows(HD), rows(HD),
                  pl.BlockSpec((NREG, _BLK, 2), lambda i: (0, i, 0)),
                  full((HD, HD)), full((HD, HD)), full((HD, HD)), full((1, HD))],
        out_specs=rows(HD),
        out_shape=jax.ShapeDtypeStruct((NND, HD), jnp.float32),
    )(x, s0, s1, cnt, root, r0, r1, bias)


def _tc_conv2(x, s0, s1, cnt, root, r0, r1, bias, wo1, bo1, wo2, bo2):
    full = lambda s: pl.BlockSpec(s, lambda i: (0, 0))
    rows = lambda w: pl.BlockSpec((_BLK, w), lambda i: (i, 0))
    return pl.pallas_call(
        _tc_conv2_body,
        grid=(_GRID,),
        in_specs=[rows(HD), rows(HD), rows(HD),
                  pl.BlockSpec((NREG, _BLK, 2), lambda i: (0, i, 0)),
                  full((HD, HD)), full((HD, HD)), full((HD, HD)), full((1, HD)),
                  full((HD, HD)), full((1, HD)), full((HD, 2)), full((1, 2))],
        out_specs=rows(2),
        out_shape=jax.ShapeDtypeStruct((NND, 2), jnp.float32),
    )(x, s0, s1, cnt, root, r0, r1, bias, wo1, bo1, wo2, bo2)


def kernel(des, tweet, num_prop, cat_prop, edge_index, edge_type,
           W_des, b_des, W_tweet, b_tweet, W_num, b_num, W_cat, b_cat,
           W_in, b_in, root1, rel1, bias1, root2, rel2, bias2,
           W_o1, b_o1, W_o2, b_o2):
    src = edge_index[0]
    dst = edge_index[1]
    et = edge_type

    npad = jnp.pad(num_prop, ((0, 0), (0, 3)))
    cpad = jnp.pad(cat_prop, ((0, 0), (0, 5)))
    wn = jnp.pad(W_num, ((0, 3), (0, 0)))
    wc = jnp.pad(W_cat, ((0, 5), (0, 0)))
    r2 = lambda b: b.reshape(1, -1)

    x = _tc_pre(des, tweet, npad, cpad, W_des, W_tweet, wn, wc,
                r2(b_des), r2(b_tweet), r2(b_num), r2(b_cat), W_in, r2(b_in))

    psrc, pdst, narr, cnts = _sc_part(src, dst, et)
    cnt = cnts.reshape(NREG, NPADR, 2)

    s1 = _sc_agg(x, psrc, pdst, narr).reshape(2, NPADR, HD)
    x1 = _tc_conv(x, s1[0], s1[1], cnt, root1, rel1[0], rel1[1], r2(bias1))

    s2 = _sc_agg(x1, psrc, pdst, narr).reshape(2, NPADR, HD)
    out = _tc_conv2(x1, s2[0], s2[1], cnt, root2, rel2[0], rel2[1], r2(bias2),
                    W_o1, r2(b_o1), W_o2, r2(b_o2))
    return out


# speedup vs baseline: 2.1610x; 2.1610x over previous
import functools

import jax
import jax.numpy as jnp
from jax import lax
from jax.experimental import pallas as pl
from jax.experimental.pallas import tpu as pltpu
from jax.experimental.pallas import tpu_sc as plsc

NND = 10000
NED = 320000
HD = 128
NCORE, NSUB, LN = 2, 16, 16
NREG = NCORE * NSUB
NPADR = 10240
ACCR = 10368
DUMPROW = NPADR
CH = 128
SL = NED // NREG
BLKE = 2000
PADN = SL + 512

_mesh = plsc.VectorSubcoreMesh(
    core_axis_name="c", subcore_axis_name="s", num_cores=NCORE, num_subcores=NSUB)
_sc_params = pltpu.CompilerParams(
    needs_layout_passes=False, use_tc_tiling_on_sc=False)


def _lrelu(v):
    return jnp.where(v >= 0, v, 0.01 * v)


@functools.partial(
    pl.kernel,
    out_type=[
        jax.ShapeDtypeStruct((2 * NREG * PADN,), jnp.int32),
        jax.ShapeDtypeStruct((2 * NREG * PADN,), jnp.int32),
        jax.ShapeDtypeStruct((NREG, LN), jnp.int32),
        jax.ShapeDtypeStruct((NREG, 2 * NPADR), jnp.float32),
    ],
    mesh=_mesh,
    scratch_types=[
        pltpu.VMEM((BLKE,), jnp.int32),
        pltpu.VMEM((BLKE,), jnp.int32),
        pltpu.VMEM((BLKE,), jnp.int32),
        pltpu.VMEM((PADN,), jnp.int32),
        pltpu.VMEM((PADN,), jnp.int32),
        pltpu.VMEM((PADN,), jnp.int32),
        pltpu.VMEM((PADN,), jnp.int32),
        pltpu.VMEM((LN,), jnp.int32),
        pltpu.VMEM((2 * NPADR,), jnp.float32),
    ],
    compiler_params=_sc_params,
)
def _sc_part(src_hbm, dst_hbm, typ_hbm, psrc_hbm, pdst_hbm, narr_hbm, cnts_hbm,
             sb, db, tb, cs0, cd0, cs1, cd1, nb, cnt):
    cid = lax.axis_index("c")
    sid = lax.axis_index("s")
    wid = cid * NSUB + sid
    zeros16 = jnp.zeros((LN,), jnp.float32)
    zeros16i = jnp.zeros((LN,), jnp.int32)
    ones16 = jnp.ones((LN,), jnp.float32)
    dump16 = jnp.full((LN,), DUMPROW, jnp.int32)
    iota16 = lax.broadcasted_iota(jnp.int32, (LN,), 0)

    def zc(i, carry):
        cnt[pl.ds(i * LN, LN)] = zeros16
        return carry
    lax.fori_loop(0, 2 * NPADR // LN, zc, 0)

    def blk(b, fills):
        eoff = wid * SL + b * BLKE
        pltpu.sync_copy(src_hbm.at[pl.ds(eoff, BLKE)], sb)
        pltpu.sync_copy(dst_hbm.at[pl.ds(eoff, BLKE)], db)
        pltpu.sync_copy(typ_hbm.at[pl.ds(eoff, BLKE)], tb)

        def grp(g, fills2):
            f0, f1 = fills2
            s16 = sb[pl.ds(g * LN, LN)]
            d16 = db[pl.ds(g * LN, LN)]
            t16 = tb[pl.ds(g * LN, LN)]
            plsc.addupdate_scatter(cnt, [d16 * 2 + t16], ones16)
            m0 = t16 == 0
            m1 = jnp.logical_not(m0)
            plsc.store_compressed(cs0.at[pl.ds(f0, LN)], s16, mask=m0)
            plsc.store_compressed(cd0.at[pl.ds(f0, LN)], d16, mask=m0)
            plsc.store_compressed(cs1.at[pl.ds(f1, LN)], s16, mask=m1)
            plsc.store_compressed(cd1.at[pl.ds(f1, LN)], d16, mask=m1)
            c0 = jnp.sum(m0.astype(jnp.int32))
            return (f0 + c0, f1 + (LN - c0))
        return lax.fori_loop(0, BLKE // LN, grp, fills)
    f0, f1 = lax.fori_loop(0, SL // BLKE, blk, (0, 0))

    for k in range(512 // LN):
        cs0[pl.ds(f0 + k * LN, LN)] = zeros16i
        cd0[pl.ds(f0 + k * LN, LN)] = dump16
        cs1[pl.ds(f1 + k * LN, LN)] = zeros16i
        cd1[pl.ds(f1 + k * LN, LN)] = dump16
    p0 = lax.shift_right_logical(f0 + 511, 9)
    p1 = lax.shift_right_logical(f1 + 511, 9)
    nb[pl.ds(0, LN)] = jnp.where(iota16 == 0, p0, jnp.where(iota16 == 1, p1, 0))

    pltpu.sync_copy(cs0, psrc_hbm.at[pl.ds(wid * PADN, PADN)])
    pltpu.sync_copy(cd0, pdst_hbm.at[pl.ds(wid * PADN, PADN)])
    pltpu.sync_copy(cs1, psrc_hbm.at[pl.ds((NREG + wid) * PADN, PADN)])
    pltpu.sync_copy(cd1, pdst_hbm.at[pl.ds((NREG + wid) * PADN, PADN)])
    pltpu.sync_copy(nb, narr_hbm.at[wid])
    pltpu.sync_copy(cnt, cnts_hbm.at[wid])


@functools.partial(
    pl.kernel,
    out_type=jax.ShapeDtypeStruct((2 * NPADR, HD), jnp.float32),
    mesh=_mesh,
    scratch_types=[
        pltpu.VMEM_SHARED((ACCR, HD), jnp.float32),
        pltpu.VMEM((LN,), jnp.int32),
        pltpu.VMEM((LN,), jnp.int32),
        pltpu.VMEM((CH,), jnp.int32),
        pltpu.VMEM((CH,), jnp.int32),
        pltpu.VMEM((CH,), jnp.int32),
        pltpu.VMEM((CH,), jnp.int32),
        pltpu.VMEM((CH,), jnp.int32),
        pltpu.VMEM((CH,), jnp.int32),
        pltpu.VMEM((CH,), jnp.int32),
        pltpu.VMEM((CH,), jnp.int32),
        pltpu.VMEM((CH, HD), jnp.float32),
        pltpu.VMEM((CH, HD), jnp.float32),
        pltpu.SemaphoreType.DMA,
        pltpu.SemaphoreType.DMA,
        pltpu.SemaphoreType.DMA,
        pltpu.SemaphoreType.DMA,
        pltpu.SemaphoreType.DMA,
        pltpu.SemaphoreType.DMA,
        pltpu.SemaphoreType.DMA,
        pltpu.SemaphoreType.DMA,
    ],
    compiler_params=_sc_params,
)
def _sc_agg(x_hbm, psrc_hbm, pdst_hbm, narr_hbm, out_hbm,
            accum, nbA, nbB, s0, d0, s1, d1, s2, d2, s3, d3,
            rowsA, rowsB, ix0, ix1, ix2, ix3, gsA, gsB, ssA, ssB):
    cid = lax.axis_index("c")
    sid = lax.axis_index("s")
    zeros16 = jnp.zeros((LN,), jnp.float32)
    dump16 = jnp.full((LN,), DUMPROW, jnp.int32)
    sets = ((s0, d0, ix0), (s1, d1, ix1), (s2, d2, ix2), (s3, d3, ix3))

    def zrow(r, carry):
        for c in range(8):
            rowsA[r, pl.ds(c * LN, LN)] = zeros16
            rowsB[r, pl.ds(c * LN, LN)] = zeros16
        return carry
    lax.fori_loop(0, CH, zrow, 0)
    for v in range(CH // LN):
        d2[pl.ds(v * LN, LN)] = dump16
        d3[pl.ds(v * LN, LN)] = dump16

    ZR = ACCR // NSUB
    def zacc(i, carry):
        pltpu.sync_copy(rowsA, accum.at[pl.ds(sid * ZR + i * CH, CH)])
        return carry
    lax.fori_loop(0, ZR // CH, zacc, 0)
    pltpu.sync_copy(rowsA.at[pl.ds(0, ZR % CH)],
                    accum.at[pl.ds(sid * ZR + (ZR // CH) * CH, ZR % CH)])
    plsc.subcore_barrier()

    pltpu.sync_copy(narr_hbm.at[2 * sid], nbA)
    pltpu.sync_copy(narr_hbm.at[2 * sid + 1], nbB)
    iota16 = lax.broadcasted_iota(jnp.int32, (LN,), 0)
    qA = jnp.sum(jnp.where(iota16 == cid, nbA[pl.ds(0, LN)], 0))
    qB = jnp.sum(jnp.where(iota16 == cid, nbB[pl.ds(0, LN)], 0))
    baseA = (cid * NREG + 2 * sid) * PADN
    baseB = (cid * NREG + 2 * sid + 1) * PADN
    tot = 4 * (qA + qB)

    def off(c):
        return pl.multiple_of(
            jnp.where(c < 4 * qA, baseA + c * CH, baseB + (c - 4 * qA) * CH), CH)

    def offpf(c):
        return pl.multiple_of(
            jnp.where(c < tot,
                      jnp.where(c < 4 * qA, baseA + c * CH,
                                baseB + (c - 4 * qA) * CH),
                      baseA), CH)

    def idx_pair(k, o):
        sb, db, sem = sets[k]
        return (pltpu.make_async_copy(psrc_hbm.at[pl.ds(o, CH)], sb, sem),
                pltpu.make_async_copy(pdst_hbm.at[pl.ds(o, CH)], db, sem))

    def fire_idx(k, o):
        a, b = idx_pair(k, o)
        a.start(); b.start()

    def wait_idx(k, o):
        a, b = idx_pair(k, o)
        a.wait(); b.wait()

    def scat(rows, idx, sem):
        return pltpu.make_async_copy(rows, accum.at[idx], sem)

    def gath(idx, rows, sem):
        return pltpu.make_async_copy(x_hbm.at[idx], rows, sem)

    fire_idx(0, offpf(0))
    fire_idx(1, offpf(1))
    scat(rowsA, d2, ssA).start(add=True)
    scat(rowsB, d3, ssB).start(add=True)

    def quad(g, carry):
        c0 = 4 * g
        wait_idx(0, off(c0))
        scat(rowsA, d2, ssA).wait()
        fire_idx(2, off(c0 + 2))
        gath(s0, rowsA, gsA).start()
        wait_idx(1, off(c0 + 1))
        scat(rowsB, d3, ssB).wait()
        fire_idx(3, off(c0 + 3))
        gath(s1, rowsB, gsB).start()
        gath(s0, rowsA, gsA).wait()
        scat(rowsA, d0, ssA).start(add=True)
        gath(s1, rowsB, gsB).wait()
        scat(rowsB, d1, ssB).start(add=True)
        scat(rowsA, d0, ssA).wait()
        fire_idx(0, offpf(c0 + 4))
        wait_idx(2, off(c0 + 2))
        gath(s2, rowsA, gsA).start()
        scat(rowsB, d1, ssB).wait()
        fire_idx(1, offpf(c0 + 5))
        wait_idx(3, off(c0 + 3))
        gath(s3, rowsB, gsB).start()
        gath(s2, rowsA, gsA).wait()
        scat(rowsA, d2, ssA).start(add=True)
        gath(s3, rowsB, gsB).wait()
        scat(rowsB, d3, ssB).start(add=True)
        return carry
    lax.fori_loop(0, qA + qB, quad, 0)
    scat(rowsA, d2, ssA).wait()
    scat(rowsB, d3, ssB).wait()
    wait_idx(0, offpf(tot))
    wait_idx(1, offpf(tot + 1))
    plsc.subcore_barrier()

    rpt = NPADR // NSUB
    pltpu.sync_copy(accum.at[pl.ds(sid * rpt, rpt)],
                    out_hbm.at[pl.ds(cid * NPADR + sid * rpt, rpt)])


_BLK = 1000
_GRID = NND // _BLK


def _tc_pre_body(des_r, tw_r, np_r, cp_r, wd_r, wt_r, wn_r, wc_r,
                 bd_r, bt_r, bn_r, bc_r, win_r, bin_r, out_r):
    d = _lrelu(jnp.dot(des_r[...], wd_r[...], preferred_element_type=jnp.float32) + bd_r[...])
    t = _lrelu(jnp.dot(tw_r[...], wt_r[...], preferred_element_type=jnp.float32) + bt_r[...])
    n = _lrelu(jnp.dot(np_r[...], wn_r[...], preferred_element_type=jnp.float32) + bn_r[...])
    c = _lrelu(jnp.dot(cp_r[...], wc_r[...], preferred_element_type=jnp.float32) + bc_r[...])
    x = jnp.concatenate([d, t, n, c], axis=1)
    out_r[...] = _lrelu(jnp.dot(x, win_r[...], preferred_element_type=jnp.float32) + bin_r[...])


def _tc_pre(des, tw, npad, cpad, wd, wt, wn, wc, bd, bt, bn, bc, win, bin_):
    full = lambda s: pl.BlockSpec(s, lambda i: (0, 0))
    rows = lambda w: pl.BlockSpec((_BLK, w), lambda i: (i, 0))
    return pl.pallas_call(
        _tc_pre_body,
        grid=(_GRID,),
        in_specs=[rows(768), rows(768), rows(8), rows(8),
                  full((768, 32)), full((768, 32)), full((8, 32)), full((8, 32)),
                  full((1, 32)), full((1, 32)), full((1, 32)), full((1, 32)),
                  full((HD, HD)), full((1, HD))],
        out_specs=rows(HD),
        out_shape=jax.ShapeDtypeStruct((NND, HD), jnp.float32),
    )(des, tw, npad, cpad, wd, wt, wn, wc, bd, bt, bn, bc, win, bin_)


def _conv_out(x_r, s0_r, s1_r, cnt_r, root_r, r0_r, r1_r, bias_r):
    cnt = jnp.sum(cnt_r[...], axis=0)
    c0 = jnp.maximum(cnt[:, 0:1], 1.0)
    c1 = jnp.maximum(cnt[:, 1:2], 1.0)
    h0 = s0_r[...] / c0
    h1 = s1_r[...] / c1
    return (jnp.dot(x_r[...], root_r[...], preferred_element_type=jnp.float32)
            + bias_r[...]
            + jnp.dot(h0, r0_r[...], preferred_element_type=jnp.float32)
            + jnp.dot(h1, r1_r[...], preferred_element_type=jnp.float32))


def _tc_conv_body(x_r, s0_r, s1_r, cnt_r, root_r, r0_r, r1_r, bias_r, out_r):
    out_r[...] = _conv_out(x_r, s0_r, s1_r, cnt_r, root_r, r0_r, r1_r, bias_r)


def _tc_conv2_body(x_r, s0_r, s1_r, cnt_r, root_r, r0_r, r1_r, bias_r,
                   wo1_r, bo1_r, wo2_r, bo2_r, out_r):
    o = _conv_out(x_r, s0_r, s1_r, cnt_r, root_r, r0_r, r1_r, bias_r)
    y = _lrelu(jnp.dot(o, wo1_r[...], preferred_element_type=jnp.float32) + bo1_r[...])
    out_r[...] = jnp.dot(y, wo2_r[...], preferred_element_type=jnp.float32) + bo2_r[...]


def _tc_conv(x, s0, s1, cnt, root, r0, r1, bias):
    full = lambda s: pl.BlockSpec(s, lambda i: (0, 0))
    rows = lambda w: pl.BlockSpec((_BLK, w), lambda i: (i, 0))
    return pl.pallas_call(
        _tc_conv_body,
        grid=(_GRID,),
        in_specs=[rows(HD), rows(HD), rows(HD),
                  pl.BlockSpec((NREG, _BLK, 2), lambda i: (0, i, 0)),
                  full((HD, HD)), full((HD, HD)), full((HD, HD)), full((1, HD))],
        out_specs=rows(HD),
        out_shape=jax.ShapeDtypeStruct((NND, HD), jnp.float32),
    )(x, s0, s1, cnt, root, r0, r1, bias)


def _tc_conv2(x, s0, s1, cnt, root, r0, r1, bias, wo1, bo1, wo2, bo2):
    full = lambda s: pl.BlockSpec(s, lambda i: (0, 0))
    rows = lambda w: pl.BlockSpec((_BLK, w), lambda i: (i, 0))
    return pl.pallas_call(
        _tc_conv2_body,
        grid=(_GRID,),
        in_specs=[rows(HD), rows(HD), rows(HD),
                  pl.BlockSpec((NREG, _BLK, 2), lambda i: (0, i, 0)),
                  full((HD, HD)), full((HD, HD)), full((HD, HD)), full((1, HD)),
                  full((HD, HD)), full((1, HD)), full((HD, 2)), full((1, 2))],
        out_specs=rows(2),
        out_shape=jax.ShapeDtypeStruct((NND, 2), jnp.float32),
    )(x, s0, s1, cnt, root, r0, r1, bias, wo1, bo1, wo2, bo2)


def kernel(des, tweet, num_prop, cat_prop, edge_index, edge_type,
           W_des, b_des, W_tweet, b_tweet, W_num, b_num, W_cat, b_cat,
           W_in, b_in, root1, rel1, bias1, root2, rel2, bias2,
           W_o1, b_o1, W_o2, b_o2):
    src = edge_index[0]
    dst = edge_index[1]
    et = edge_type

    npad = jnp.pad(num_prop, ((0, 0), (0, 3)))
    cpad = jnp.pad(cat_prop, ((0, 0), (0, 5)))
    wn = jnp.pad(W_num, ((0, 3), (0, 0)))
    wc = jnp.pad(W_cat, ((0, 5), (0, 0)))
    r2 = lambda b: b.reshape(1, -1)

    x = _tc_pre(des, tweet, npad, cpad, W_des, W_tweet, wn, wc,
                r2(b_des), r2(b_tweet), r2(b_num), r2(b_cat), W_in, r2(b_in))

    psrc, pdst, narr, cnts = _sc_part(src, dst, et)
    cnt = cnts.reshape(NREG, NPADR, 2)

    s1 = _sc_agg(x, psrc, pdst, narr).reshape(2, NPADR, HD)
    x1 = _tc_conv(x, s1[0], s1[1], cnt, root1, rel1[0], rel1[1], r2(bias1))

    s2 = _sc_agg(x1, psrc, pdst, narr).reshape(2, NPADR, HD)
    out = _tc_conv2(x1, s2[0], s2[1], cnt, root2, rel2[0], rel2[1], r2(bias2),
                    W_o1, r2(b_o1), W_o2, r2(b_o2))
    return out
